# baseline (speedup 1.0000x reference)
import functools

import numpy as np

import jax
import jax.numpy as jnp
from jax import lax
from jax.experimental import pallas as pl
from jax.experimental.pallas import tpu as pltpu
from jax.experimental.pallas import tpu_sc as plsc

N = 10000
E = 160000
D = 256
H = 128
HW = H // 2
NC = 2
NS = 16
EPT = E // NS
CH = 96
NFULL = EPT // CH
TAIL = EPT - NFULL * CH
RPT = (N // NS) // 8 * 8
REXTRA = N - NS * RPT

_PERM = np.concatenate(
    [np.concatenate([np.arange(b * 32, b * 32 + 32)[0::2],
                     np.arange(b * 32, b * 32 + 32)[1::2]])
     for b in range(D // 32)])

_MASK_HI = np.int32(-65536)


def _sc_agg_body(x_hbm, xb_hbm, src_hbm, dst_hbm, out_hbm, src0, dst0, rows0,
                 src1, dst1, rows1, src2, dst2, rows2, fbuf, tsrc, tdst, trows,
                 acc, sem0, sem1, sem2):
    c = lax.axis_index("c")
    s = lax.axis_index("s")
    roff = c * N

    r0 = s * RPT
    pltpu.sync_copy(x_hbm.at[pl.ds(roff + r0, RPT)], acc.at[pl.ds(r0, RPT)])

    @pl.when(s == NS - 1)
    def _init_extra():
        pltpu.sync_copy(x_hbm.at[pl.ds(roff + NS * RPT, REXTRA)],
                        acc.at[pl.ds(NS * RPT, REXTRA)])

    plsc.subcore_barrier()

    ebase = s * EPT

    def start(srcb, dstb, rowsb, semb, e0):
        pltpu.sync_copy(src_hbm.at[pl.ds(e0, CH)], srcb)
        pltpu.sync_copy(dst_hbm.at[pl.ds(e0, CH)], dstb)
        for i in range(CH // 16):
            sl = pl.ds(i * 16, 16)
            srcb[sl] = srcb[sl] + roff
        pltpu.async_copy(xb_hbm.at[srcb], rowsb, semb)

    def convert(rowsb, nrows):
        @pl.loop(0, nrows, unroll=4)
        def _rows(r):
            for m in range(HW // 16):
                w = rowsb[r, pl.ds(m * 16, 16)]
                lo = lax.bitcast_convert_type(lax.shift_left(w, 16),
                                              jnp.float32)
                hi = lax.bitcast_convert_type(w & _MASK_HI, jnp.float32)
                fbuf[r, pl.ds(m * 32, 16)] = lo
                fbuf[r, pl.ds(m * 32 + 16, 16)] = hi

    def finish(srcb, dstb, rowsb, semb):
        pltpu.make_async_copy(xb_hbm.at[srcb], rowsb, semb).wait()
        convert(rowsb, CH)
        pltpu.sync_copy(fbuf, acc.at[dstb], add=True)

    start(src0, dst0, rows0, sem0, ebase)
    start(src1, dst1, rows1, sem1, ebase + CH)

    @pl.loop(0, NFULL // 3)
    def _chunks(t):
        e0 = ebase + t * (3 * CH)
        pltpu.make_async_copy(xb_hbm.at[src0], rows0, sem0).wait()
        start(src2, dst2, rows2, sem2, e0 + 2 * CH)
        convert(rows0, CH)
        pltpu.sync_copy(fbuf, acc.at[dst0], add=True)

        pltpu.make_async_copy(xb_hbm.at[src1], rows1, sem1).wait()
        start(src0, dst0, rows0, sem0, e0 + 3 * CH)
        convert(rows1, CH)
        pltpu.sync_copy(fbuf, acc.at[dst1], add=True)

        pltpu.make_async_copy(xb_hbm.at[src2], rows2, sem2).wait()
        start(src1, dst1, rows1, sem1, e0 + 4 * CH)
        convert(rows2, CH)
        pltpu.sync_copy(fbuf, acc.at[dst2], add=True)

    finish(src0, dst0, rows0, sem0)
    finish(src1, dst1, rows1, sem1)

    if TAIL:
        e0 = ebase + NFULL * CH
        pltpu.sync_copy(src_hbm.at[pl.ds(e0, TAIL)], tsrc)
        pltpu.sync_copy(dst_hbm.at[pl.ds(e0, TAIL)], tdst)
        for i in range(TAIL // 16):
            sl = pl.ds(i * 16, 16)
            tsrc[sl] = tsrc[sl] + roff
        pltpu.async_copy(xb_hbm.at[tsrc], trows, sem0).wait()
        convert(trows, TAIL)
        pltpu.sync_copy(fbuf.at[pl.ds(0, TAIL)], acc.at[tdst], add=True)

    plsc.subcore_barrier()
    pltpu.sync_copy(acc.at[pl.ds(r0, RPT)], out_hbm.at[pl.ds(roff + r0, RPT)])

    @pl.when(s == NS - 1)
    def _out_extra():
        pltpu.sync_copy(acc.at[pl.ds(NS * RPT, REXTRA)],
                        out_hbm.at[pl.ds(roff + NS * RPT, REXTRA)])


@jax.jit
def _sc_agg(x_cat, xb_cat, src, dst):
    mesh = plsc.VectorSubcoreMesh(core_axis_name="c", subcore_axis_name="s")
    return pl.kernel(
        _sc_agg_body,
        out_type=jax.ShapeDtypeStruct((2 * N, H), jnp.float32),
        mesh=mesh,
        compiler_params=pltpu.CompilerParams(use_tc_tiling_on_sc=False),
        scratch_types=[
            pltpu.VMEM((CH,), jnp.int32),
            pltpu.VMEM((CH,), jnp.int32),
            pltpu.VMEM((CH, HW), jnp.int32),
            pltpu.VMEM((CH,), jnp.int32),
            pltpu.VMEM((CH,), jnp.int32),
            pltpu.VMEM((CH, HW), jnp.int32),
            pltpu.VMEM((CH,), jnp.int32),
            pltpu.VMEM((CH,), jnp.int32),
            pltpu.VMEM((CH, HW), jnp.int32),
            pltpu.VMEM((CH, H), jnp.float32),
            pltpu.VMEM((max(TAIL, 16),), jnp.int32),
            pltpu.VMEM((max(TAIL, 16),), jnp.int32),
            pltpu.VMEM((max(TAIL, 16), HW), jnp.int32),
            pltpu.VMEM_SHARED((N, H), jnp.float32),
            pltpu.SemaphoreType.DMA,
            pltpu.SemaphoreType.DMA,
            pltpu.SemaphoreType.DMA,
        ],
    )(x_cat, xb_cat, src, dst)


def _pack_bf16(o):
    ob = o.astype(jnp.bfloat16)
    words = []
    for m in range(D // 32):
        lo = lax.bitcast_convert_type(ob[:, m * 32:m * 32 + 16], jnp.uint16)
        hi = lax.bitcast_convert_type(ob[:, m * 32 + 16:m * 32 + 32],
                                      jnp.uint16)
        w = (lo.astype(jnp.int32)
             | lax.shift_left(hi.astype(jnp.int32), 16))
        words.append(w)
    return jnp.concatenate(words, axis=1)


def _dense_body_split(hin_ref, w_ref, b_ref, out_ref, outb_ref):
    hl = hin_ref[0]
    hh = hin_ref[1]
    h = (jnp.dot(hl, w_ref[:H, :], preferred_element_type=jnp.float32)
         + jnp.dot(hh, w_ref[H:, :], preferred_element_type=jnp.float32)
         + b_ref[...])
    o = h * jax.nn.sigmoid(h)
    out_ref[0] = o[:, :H]
    out_ref[1] = o[:, H:]
    ow = _pack_bf16(o)
    outb_ref[0] = ow[:, :HW]
    outb_ref[1] = ow[:, HW:]


def _dense_body_last(hin_ref, w_ref, b_ref, out_ref):
    hl = hin_ref[0]
    hh = hin_ref[1]
    h = (jnp.dot(hl, w_ref[:H, :], preferred_element_type=jnp.float32)
         + jnp.dot(hh, w_ref[H:, :], preferred_element_type=jnp.float32)
         + b_ref[...])
    out_ref[...] = h * jax.nn.sigmoid(h)


_RB = 2000


@functools.partial(jax.jit, static_argnames=("last",))
def _dense(hin2, w, b2, last=False):
    grid = (N // _RB,)
    in_specs = [
        pl.BlockSpec((2, _RB, H), lambda i: (0, i, 0)),
        pl.BlockSpec((D, D), lambda i: (0, 0)),
        pl.BlockSpec((1, D), lambda i: (0, 0)),
    ]
    if last:
        return pl.pallas_call(
            _dense_body_last,
            grid=grid,
            in_specs=in_specs,
            out_specs=pl.BlockSpec((_RB, D), lambda i: (i, 0)),
            out_shape=jax.ShapeDtypeStruct((N, D), jnp.float32),
        )(hin2, w, b2)
    return pl.pallas_call(
        _dense_body_split,
        grid=grid,
        in_specs=in_specs,
        out_specs=[
            pl.BlockSpec((2, _RB, H), lambda i: (0, i, 0)),
            pl.BlockSpec((2, _RB, HW), lambda i: (0, i, 0)),
        ],
        out_shape=[
            jax.ShapeDtypeStruct((2, N, H), jnp.float32),
            jax.ShapeDtypeStruct((2, N, HW), jnp.int32),
        ],
    )(hin2, w, b2)


def kernel(x, edge_index, W0, b0, W1, b1, W2, b2):
    edges = edge_index.astype(jnp.int32)
    src = edges[0]
    dst = edges[1]
    perm = _PERM
    xq = jnp.take(x, perm, axis=1)
    h2 = xq.reshape(N, 2, H).transpose(1, 0, 2)
    hb2 = _pack_bf16(xq).reshape(N, 2, HW).transpose(1, 0, 2)
    params = [(W0, b0), (W1, b1), (W2, b2)]
    for li, (w, b) in enumerate(params):
        last = li == 2
        wp = jnp.take(w, perm, axis=0)
        if not last:
            wp = jnp.take(wp, perm, axis=1)
            b = jnp.take(b, perm)
        hin = _sc_agg(h2.reshape(2 * N, H), hb2.reshape(2 * N, HW), src, dst)
        out = _dense(hin.reshape(2, N, H), wp, b.reshape(1, D), last=last)
        if not last:
            h2, hb2 = out
        else:
            h2 = out
    return h2


# speedup vs baseline: 2.0758x; 2.0758x over previous
import functools

import jax
import jax.numpy as jnp
from jax import lax
from jax.experimental import pallas as pl
from jax.experimental.pallas import tpu as pltpu
from jax.experimental.pallas import tpu_sc as plsc

N = 10000
E = 160000
D = 256
H = 128
NC = 2
NS = 16
EPT = E // NS
CH = 96
NFULL = EPT // CH
TAIL = EPT - NFULL * CH
RPT = (N // NS) // 8 * 8
REXTRA = N - NS * RPT


def _sc_agg_body(x_hbm, src_hbm, dst_hbm, out_hbm, src0, dst0, rows0, src1,
                 dst1, rows1, src2, dst2, rows2, tsrc, tdst, trows, acc, sem0,
                 sem1, sem2):
    c = lax.axis_index("c")
    s = lax.axis_index("s")
    roff = c * N

    r0 = s * RPT
    pltpu.sync_copy(x_hbm.at[pl.ds(roff + r0, RPT)], acc.at[pl.ds(r0, RPT)])

    @pl.when(s == NS - 1)
    def _init_extra():
        pltpu.sync_copy(x_hbm.at[pl.ds(roff + NS * RPT, REXTRA)],
                        acc.at[pl.ds(NS * RPT, REXTRA)])

    plsc.subcore_barrier()

    ebase = s * EPT

    def start(srcb, dstb, rowsb, semb, e0):
        pltpu.sync_copy(src_hbm.at[pl.ds(e0, CH)], srcb)
        pltpu.sync_copy(dst_hbm.at[pl.ds(e0, CH)], dstb)
        for i in range(CH // 16):
            sl = pl.ds(i * 16, 16)
            srcb[sl] = srcb[sl] + roff
        pltpu.async_copy(x_hbm.at[srcb], rowsb, semb)

    def finish(srcb, dstb, rowsb, semb):
        pltpu.make_async_copy(x_hbm.at[srcb], rowsb, semb).wait()
        pltpu.sync_copy(rowsb, acc.at[dstb], add=True)

    start(src0, dst0, rows0, sem0, ebase)
    start(src1, dst1, rows1, sem1, ebase + CH)

    @pl.loop(0, NFULL // 3)
    def _chunks(t):
        e0 = ebase + t * (3 * CH)
        pltpu.make_async_copy(x_hbm.at[src0], rows0, sem0).wait()
        start(src2, dst2, rows2, sem2, e0 + 2 * CH)
        pltpu.sync_copy(rows0, acc.at[dst0], add=True)

        pltpu.make_async_copy(x_hbm.at[src1], rows1, sem1).wait()
        start(src0, dst0, rows0, sem0, e0 + 3 * CH)
        pltpu.sync_copy(rows1, acc.at[dst1], add=True)

        pltpu.make_async_copy(x_hbm.at[src2], rows2, sem2).wait()
        start(src1, dst1, rows1, sem1, e0 + 4 * CH)
        pltpu.sync_copy(rows2, acc.at[dst2], add=True)

    finish(src0, dst0, rows0, sem0)
    finish(src1, dst1, rows1, sem1)

    if TAIL:
        e0 = ebase + NFULL * CH
        pltpu.sync_copy(src_hbm.at[pl.ds(e0, TAIL)], tsrc)
        pltpu.sync_copy(dst_hbm.at[pl.ds(e0, TAIL)], tdst)
        for i in range(TAIL // 16):
            sl = pl.ds(i * 16, 16)
            tsrc[sl] = tsrc[sl] + roff
        pltpu.async_copy(x_hbm.at[tsrc], trows, sem0).wait()
        pltpu.sync_copy(trows, acc.at[tdst], add=True)

    plsc.subcore_barrier()
    pltpu.sync_copy(acc.at[pl.ds(r0, RPT)], out_hbm.at[pl.ds(roff + r0, RPT)])

    @pl.when(s == NS - 1)
    def _out_extra():
        pltpu.sync_copy(acc.at[pl.ds(NS * RPT, REXTRA)],
                        out_hbm.at[pl.ds(roff + NS * RPT, REXTRA)])


@jax.jit
def _sc_agg(x_cat, src, dst):
    mesh = plsc.VectorSubcoreMesh(core_axis_name="c", subcore_axis_name="s")
    return pl.kernel(
        _sc_agg_body,
        out_type=jax.ShapeDtypeStruct((2 * N, H), jnp.float32),
        mesh=mesh,
        scratch_types=[
            pltpu.VMEM((CH,), jnp.int32),
            pltpu.VMEM((CH,), jnp.int32),
            pltpu.VMEM((CH, H), jnp.float32),
            pltpu.VMEM((CH,), jnp.int32),
            pltpu.VMEM((CH,), jnp.int32),
            pltpu.VMEM((CH, H), jnp.float32),
            pltpu.VMEM((CH,), jnp.int32),
            pltpu.VMEM((CH,), jnp.int32),
            pltpu.VMEM((CH, H), jnp.float32),
            pltpu.VMEM((max(TAIL, 16),), jnp.int32),
            pltpu.VMEM((max(TAIL, 16),), jnp.int32),
            pltpu.VMEM((max(TAIL, 16), H), jnp.float32),
            pltpu.VMEM_SHARED((N, H), jnp.float32),
            pltpu.SemaphoreType.DMA,
            pltpu.SemaphoreType.DMA,
            pltpu.SemaphoreType.DMA,
        ],
    )(x_cat, src, dst)


def _dense_body_split(hin_ref, w_ref, b_ref, out_ref):
    hl = hin_ref[0]
    hh = hin_ref[1]
    h = (jnp.dot(hl, w_ref[:H, :], preferred_element_type=jnp.float32)
         + jnp.dot(hh, w_ref[H:, :], preferred_element_type=jnp.float32)
         + b_ref[...])
    o = h * jax.nn.sigmoid(h)
    out_ref[0] = o[:, :H]
    out_ref[1] = o[:, H:]


def _dense_body_last(hin_ref, w_ref, b_ref, out_ref):
    hl = hin_ref[0]
    hh = hin_ref[1]
    h = (jnp.dot(hl, w_ref[:H, :], preferred_element_type=jnp.float32)
         + jnp.dot(hh, w_ref[H:, :], preferred_element_type=jnp.float32)
         + b_ref[...])
    out_ref[...] = h * jax.nn.sigmoid(h)


_RB = 2000


@functools.partial(jax.jit, static_argnames=("last",))
def _dense(hin2, w, b2, last=False):
    grid = (N // _RB,)
    in_specs = [
        pl.BlockSpec((2, _RB, H), lambda i: (0, i, 0)),
        pl.BlockSpec((D, D), lambda i: (0, 0)),
        pl.BlockSpec((1, D), lambda i: (0, 0)),
    ]
    if last:
        return pl.pallas_call(
            _dense_body_last,
            grid=grid,
            in_specs=in_specs,
            out_specs=pl.BlockSpec((_RB, D), lambda i: (i, 0)),
            out_shape=jax.ShapeDtypeStruct((N, D), jnp.float32),
        )(hin2, w, b2)
    return pl.pallas_call(
        _dense_body_split,
        grid=grid,
        in_specs=in_specs,
        out_specs=pl.BlockSpec((2, _RB, H), lambda i: (0, i, 0)),
        out_shape=jax.ShapeDtypeStruct((2, N, H), jnp.float32),
    )(hin2, w, b2)


def kernel(x, edge_index, W0, b0, W1, b1, W2, b2):
    edges = edge_index.astype(jnp.int32)
    src = edges[0]
    dst = edges[1]
    h2 = x.reshape(N, 2, H).transpose(1, 0, 2)
    params = [(W0, b0), (W1, b1), (W2, b2)]
    for li, (w, b) in enumerate(params):
        hin = _sc_agg(h2.reshape(2 * N, H), src, dst)
        h2 = _dense(hin.reshape(2, N, H), w, b.reshape(1, D), last=(li == 2))
    return h2


# speedup vs baseline: 2.2054x; 1.0625x over previous
import functools

import jax
import jax.numpy as jnp
from jax import lax
from jax.experimental import pallas as pl
from jax.experimental.pallas import tpu as pltpu
from jax.experimental.pallas import tpu_sc as plsc

N = 10000
E = 160000
D = 256
H = 128
NC = 2
NS = 16
EPT = E // NS
CH = 112
NFULL = EPT // CH
TAIL = EPT - NFULL * CH
RPT = (N // NS) // 8 * 8
REXTRA = N - NS * RPT


def _sc_agg_body(x_hbm, src_hbm, dst_hbm, out_hbm, src0, dst0, rows0, src1,
                 dst1, rows1, src2, dst2, rows2, tsrc, tdst, trows, acc, sem0,
                 sem1, sem2):
    c = lax.axis_index("c")
    s = lax.axis_index("s")
    roff = c * N

    r0 = s * RPT
    pltpu.sync_copy(x_hbm.at[pl.ds(roff + r0, RPT)], acc.at[pl.ds(r0, RPT)])

    @pl.when(s == NS - 1)
    def _init_extra():
        pltpu.sync_copy(x_hbm.at[pl.ds(roff + NS * RPT, REXTRA)],
                        acc.at[pl.ds(NS * RPT, REXTRA)])

    plsc.subcore_barrier()

    ebase = s * EPT

    def start(srcb, dstb, rowsb, semb, e0):
        pltpu.sync_copy(src_hbm.at[pl.ds(e0, CH)], srcb)
        pltpu.sync_copy(dst_hbm.at[pl.ds(e0, CH)], dstb)
        for i in range(CH // 16):
            sl = pl.ds(i * 16, 16)
            srcb[sl] = srcb[sl] + roff
        pltpu.async_copy(x_hbm.at[srcb], rowsb, semb)

    def finish(srcb, dstb, rowsb, semb):
        pltpu.make_async_copy(x_hbm.at[srcb], rowsb, semb).wait()
        pltpu.sync_copy(rowsb, acc.at[dstb], add=True)

    start(src0, dst0, rows0, sem0, ebase)
    start(src1, dst1, rows1, sem1, ebase + CH)

    @pl.loop(0, NFULL // 3)
    def _chunks(t):
        e0 = ebase + t * (3 * CH)
        pltpu.make_async_copy(x_hbm.at[src0], rows0, sem0).wait()
        start(src2, dst2, rows2, sem2, e0 + 2 * CH)
        pltpu.sync_copy(rows0, acc.at[dst0], add=True)

        pltpu.make_async_copy(x_hbm.at[src1], rows1, sem1).wait()
        start(src0, dst0, rows0, sem0, e0 + 3 * CH)
        pltpu.sync_copy(rows1, acc.at[dst1], add=True)

        pltpu.make_async_copy(x_hbm.at[src2], rows2, sem2).wait()
        start(src1, dst1, rows1, sem1, e0 + 4 * CH)
        pltpu.sync_copy(rows2, acc.at[dst2], add=True)

    finish(src0, dst0, rows0, sem0)
    finish(src1, dst1, rows1, sem1)

    if TAIL:
        e0 = ebase + NFULL * CH
        pltpu.sync_copy(src_hbm.at[pl.ds(e0, TAIL)], tsrc)
        pltpu.sync_copy(dst_hbm.at[pl.ds(e0, TAIL)], tdst)
        for i in range(TAIL // 16):
            sl = pl.ds(i * 16, 16)
            tsrc[sl] = tsrc[sl] + roff
        pltpu.async_copy(x_hbm.at[tsrc], trows, sem0).wait()
        pltpu.sync_copy(trows, acc.at[tdst], add=True)

    plsc.subcore_barrier()
    pltpu.sync_copy(acc.at[pl.ds(r0, RPT)], out_hbm.at[pl.ds(roff + r0, RPT)])

    @pl.when(s == NS - 1)
    def _out_extra():
        pltpu.sync_copy(acc.at[pl.ds(NS * RPT, REXTRA)],
                        out_hbm.at[pl.ds(roff + NS * RPT, REXTRA)])


@jax.jit
def _sc_agg(x_cat, src, dst):
    mesh = plsc.VectorSubcoreMesh(core_axis_name="c", subcore_axis_name="s")
    return pl.kernel(
        _sc_agg_body,
        out_type=jax.ShapeDtypeStruct((2 * N, H), jnp.float32),
        mesh=mesh,
        scratch_types=[
            pltpu.VMEM((CH,), jnp.int32),
            pltpu.VMEM((CH,), jnp.int32),
            pltpu.VMEM((CH, H), jnp.float32),
            pltpu.VMEM((CH,), jnp.int32),
            pltpu.VMEM((CH,), jnp.int32),
            pltpu.VMEM((CH, H), jnp.float32),
            pltpu.VMEM((CH,), jnp.int32),
            pltpu.VMEM((CH,), jnp.int32),
            pltpu.VMEM((CH, H), jnp.float32),
            pltpu.VMEM((max(TAIL, 16),), jnp.int32),
            pltpu.VMEM((max(TAIL, 16),), jnp.int32),
            pltpu.VMEM((max(TAIL, 16), H), jnp.float32),
            pltpu.VMEM_SHARED((N, H), jnp.float32),
            pltpu.SemaphoreType.DMA,
            pltpu.SemaphoreType.DMA,
            pltpu.SemaphoreType.DMA,
        ],
    )(x_cat, src, dst)


def _dense_body_split(hin_ref, w_ref, b_ref, out_ref):
    hl = hin_ref[0]
    hh = hin_ref[1]
    h = (jnp.dot(hl, w_ref[:H, :], preferred_element_type=jnp.float32)
         + jnp.dot(hh, w_ref[H:, :], preferred_element_type=jnp.float32)
         + b_ref[...])
    o = h * jax.nn.sigmoid(h)
    out_ref[0] = o[:, :H]
    out_ref[1] = o[:, H:]


def _dense_body_last(hin_ref, w_ref, b_ref, out_ref):
    hl = hin_ref[0]
    hh = hin_ref[1]
    h = (jnp.dot(hl, w_ref[:H, :], preferred_element_type=jnp.float32)
         + jnp.dot(hh, w_ref[H:, :], preferred_element_type=jnp.float32)
         + b_ref[...])
    out_ref[...] = h * jax.nn.sigmoid(h)


_RB = 2000


@functools.partial(jax.jit, static_argnames=("last",))
def _dense(hin2, w, b2, last=False):
    grid = (N // _RB,)
    in_specs = [
        pl.BlockSpec((2, _RB, H), lambda i: (0, i, 0)),
        pl.BlockSpec((D, D), lambda i: (0, 0)),
        pl.BlockSpec((1, D), lambda i: (0, 0)),
    ]
    if last:
        return pl.pallas_call(
            _dense_body_last,
            grid=grid,
            in_specs=in_specs,
            out_specs=pl.BlockSpec((_RB, D), lambda i: (i, 0)),
            out_shape=jax.ShapeDtypeStruct((N, D), jnp.float32),
        )(hin2, w, b2)
    return pl.pallas_call(
        _dense_body_split,
        grid=grid,
        in_specs=in_specs,
        out_specs=pl.BlockSpec((2, _RB, H), lambda i: (0, i, 0)),
        out_shape=jax.ShapeDtypeStruct((2, N, H), jnp.float32),
    )(hin2, w, b2)


def kernel(x, edge_index, W0, b0, W1, b1, W2, b2):
    edges = edge_index.astype(jnp.int32)
    src = edges[0]
    dst = edges[1]
    h2 = x.reshape(N, 2, H).transpose(1, 0, 2)
    params = [(W0, b0), (W1, b1), (W2, b2)]
    for li, (w, b) in enumerate(params):
        hin = _sc_agg(h2.reshape(2 * N, H), src, dst)
        h2 = _dense(hin.reshape(2, N, H), w, b.reshape(1, D), last=(li == 2))
    return h2
